# baseline (speedup 1.0000x reference)
import functools

import jax
import jax.numpy as jnp
from jax import lax
from jax.experimental import pallas as pl
from jax.experimental.pallas import tpu as pltpu
from jax.experimental.pallas import tpu_sc as plsc

_VOCAB = 100000
_EMBD = 128
_CTX = 10
_HID = 512
_BN = 1280
_NC = _VOCAB // _BN
_TAIL = _VOCAB - _NC * _BN
_NBUF = 13
_NSTEP = _NC // _NBUF


def _sc_gather(idx, emb):
    n = idx.shape[0]
    mesh = plsc.VectorSubcoreMesh(core_axis_name="c", subcore_axis_name="s")

    @functools.partial(
        pl.kernel,
        out_type=jax.ShapeDtypeStruct((n, _EMBD), jnp.float32),
        mesh=mesh,
        scratch_types=[
            pltpu.VMEM((n,), jnp.int32),
            pltpu.VMEM((n, _EMBD), jnp.float32),
            pltpu.SemaphoreType.DMA,
        ],
    )
    def k(idx_hbm, emb_hbm, out_hbm, idx_v, rows_v, sem):
        c = lax.axis_index("c")
        s = lax.axis_index("s")

        @pl.when(jnp.logical_and(c == 0, s == 0))
        def _():
            pltpu.sync_copy(idx_hbm, idx_v)
            pltpu.async_copy(emb_hbm.at[idx_v], rows_v, sem).wait()
            pltpu.sync_copy(rows_v, out_hbm)

    return k(idx, emb)


def _chunk_copy(w2t_any, buf_s, sems, c, j):
    return pltpu.make_async_copy(
        w2t_any.at[pl.ds(c * _BN, _BN), :],
        buf_s.at[j],
        sems.at[j],
    )


def _dotT(hid, chunk):
    return lax.dot_general(hid, chunk, (((1,), (1,)), ((), ())),
                           preferred_element_type=jnp.float32)


def _mega_body(e_ref, w1_ref, b1_ref, b2_ref, w2t_any, out_ref,
               hid_s, out_s, buf_s, tail_s, sems, tail_sem):
    for j in range(_NBUF):
        _chunk_copy(w2t_any, buf_s, sems, j, j).start()
    pltpu.make_async_copy(
        w2t_any.at[pl.ds(_NC * _BN, _TAIL), :], tail_s, tail_sem).start()

    h = jnp.dot(e_ref[...], w1_ref[...], preferred_element_type=jnp.float32)
    hid_s[...] = jnp.maximum(h + b1_ref[...], 0.0)

    def step(s, carry):
        m0, s0 = carry
        for j in range(_NBUF):
            c = _NBUF * s + j
            _chunk_copy(w2t_any, buf_s, sems, c, j).wait()
            blk = _dotT(hid_s[...], buf_s[j])
            blk = blk + b2_ref[:, pl.ds(c * _BN, _BN)]
            out_s[:, pl.ds(c * _BN, _BN)] = blk
            m1 = jnp.maximum(m0, jnp.max(blk))
            s0 = s0 * jnp.exp(m0 - m1) + jnp.sum(jnp.exp(blk - m1))
            m0 = m1

            @pl.when(c + _NBUF < _NC)
            def _():
                _chunk_copy(w2t_any, buf_s, sems, c + _NBUF, j).start()
        return m0, s0

    m0, s0 = lax.fori_loop(
        0, _NSTEP, step, (jnp.float32(-jnp.inf), jnp.float32(0.0)))

    pltpu.make_async_copy(
        w2t_any.at[pl.ds(_NC * _BN, _TAIL), :], tail_s, tail_sem).wait()
    blk = _dotT(hid_s[...], tail_s[...])
    blk = blk + b2_ref[:, pl.ds(_NC * _BN, _TAIL)]
    out_s[:, pl.ds(_NC * _BN, _TAIL)] = blk
    m1 = jnp.maximum(m0, jnp.max(blk))
    s1 = s0 * jnp.exp(m0 - m1) + jnp.sum(jnp.exp(blk - m1))

    lse = m1 + jnp.log(s1)
    out_ref[...] = out_s[...] - lse


def _tc_mlp(embedded, W1, b1_row, W2T, b2_row):
    return pl.pallas_call(
        _mega_body,
        in_specs=[
            pl.BlockSpec(memory_space=pltpu.VMEM),
            pl.BlockSpec(memory_space=pltpu.VMEM),
            pl.BlockSpec(memory_space=pltpu.VMEM),
            pl.BlockSpec(memory_space=pltpu.VMEM),
            pl.BlockSpec(memory_space=pl.ANY),
        ],
        out_specs=pl.BlockSpec(memory_space=pltpu.VMEM),
        out_shape=jax.ShapeDtypeStruct((1, _VOCAB), jnp.float32),
        scratch_shapes=[
            pltpu.VMEM((1, _HID), jnp.float32),
            pltpu.VMEM((1, _VOCAB), jnp.float32),
            pltpu.VMEM((_NBUF, _BN, _HID), jnp.float32),
            pltpu.VMEM((_TAIL, _HID), jnp.float32),
            pltpu.SemaphoreType.DMA((_NBUF,)),
            pltpu.SemaphoreType.DMA,
        ],
    )(embedded, W1, b1_row, b2_row, W2T)


def kernel(inputs, emb, W1, b1, W2, b2):
    embedded = _sc_gather(inputs, emb).reshape(1, 2 * _CTX * _EMBD)
    return _tc_mlp(embedded, W1, b1.reshape(1, _HID), jnp.swapaxes(W2, 0, 1),
                   b2.reshape(1, _VOCAB))
